# baseline (speedup 1.0000x reference)
import functools

import numpy as np

import jax
import jax.numpy as jnp
from jax import lax
from jax.experimental import pallas as pl
from jax.experimental.pallas import tpu as pltpu
from jax.experimental.pallas import tpu_sc as plsc

N_ITEM = 100000
DIM = 64
NSAMP = 16
NU = 1024
HIST = 50
HPAD = 64
NC, NS, L = 2, 16, 16
NW = NC * NS
UPW = NU // NW
E1W = UPW * NSAMP
E2W = E1W * NSAMP
CH = 128


def _row_f32x4(buf, r):
    a16, b16 = plsc.unpack(buf[r, :], format=plsc.PackFormat.INTERLEAVED,
                           preferred_element_type=jnp.bfloat16)
    aa, ab = plsc.unpack(a16, format=plsc.PackFormat.INTERLEAVED)
    ba, bb = plsc.unpack(b16, format=plsc.PackFormat.INTERLEAVED)
    return (aa, ab, ba, bb)


def _sum_rows(buf, r0, n):
    parts = []
    for p in range(2):
        acc = list(_row_f32x4(buf, r0 + p))
        for k in range(p + 2, n, 2):
            x = _row_f32x4(buf, r0 + k)
            for j in range(4):
                acc[j] = acc[j] + x[j]
        parts.append(acc)
    return [parts[0][j] + parts[1][j] for j in range(4)]


NB = 4


def _sc_body(user_hbm, item_hbm, nidx_hbm, adj_hbm, emb_hbm,
             usum_out, v0_out, v1_out, m2_out,
             uidx, nbuf, ibuf, v0buf, adj1, idx1, adj2, gidx,
             g0, g1, g2, g3, vb0, vb1, usum, macc,
             s0, s1, s2, s3, sv0, sv1, sa, sb):
    bufs = [g0, g1, g2, g3]
    sems = [s0, s1, s2, s3]
    vbufs = [vb0, vb1]
    vsems = [sv0, sv1]
    wid = lax.axis_index("s") * NC + lax.axis_index("c")
    base = wid * UPW

    pltpu.sync_copy(user_hbm.at[pl.ds(base, UPW)], uidx)
    pltpu.sync_copy(nidx_hbm.at[pl.ds(base, UPW)], nbuf)
    pltpu.sync_copy(item_hbm.at[pl.ds(base, UPW)], ibuf)

    pltpu.async_copy(emb_hbm.at[ibuf], v0buf, sa)
    pltpu.async_copy(adj_hbm.at[ibuf], adj1, sa)

    lanes = lax.iota(jnp.int32, L)

    def _clamp(u, _):
        for j in range(HPAD // L):
            raw = uidx[u, pl.ds(j * L, L)]
            n_u = nbuf[u, pl.ds(j * L, L)]
            pos = lanes + (j * L)
            gidx[pl.ds(u * HPAD + j * L, L)] = jnp.where(
                pos < n_u, raw, jnp.int32(N_ITEM))
        return _
    lax.fori_loop(0, UPW, _clamp, 0, unroll=4)

    def _start(c, b):
        pltpu.async_copy(emb_hbm.at[gidx.at[pl.ds(c * CH, CH)]],
                         bufs[b], sems[b])

    for b in range(NB):
        _start(b, b)

    pltpu.make_async_copy(emb_hbm.at[ibuf], v0buf, sa).wait()
    pltpu.make_async_copy(adj_hbm.at[ibuf], adj1, sa).wait()

    def _flat1(u, _):
        idx1[pl.ds(u * NSAMP, NSAMP)] = adj1[u, :]
        return _
    lax.fori_loop(0, UPW, _flat1, 0, unroll=8)

    for c in range(E1W // CH):
        pltpu.async_copy(adj_hbm.at[idx1.at[pl.ds(c * CH, CH)]],
                         adj2.at[pl.ds(c * CH, CH)], sb)
    pltpu.sync_copy(v0buf, v0_out.at[pl.ds(base, UPW)])

    with jax.named_scope("v1_phase"):
        for c in range(2):
            pltpu.async_copy(emb_hbm.at[idx1.at[pl.ds(c * CH, CH)]],
                             vbufs[c], vsems[c])
        for c in range(E1W // CH):
            pltpu.make_async_copy(emb_hbm.at[idx1.at[pl.ds(c * CH, CH)]],
                                  vbufs[c % 2], vsems[c % 2]).wait()
            pltpu.sync_copy(vbufs[c % 2],
                            v1_out.at[pl.ds(wid * E1W + c * CH, CH)])
            if c + 2 < E1W // CH:
                pltpu.async_copy(
                    emb_hbm.at[idx1.at[pl.ds((c + 2) * CH, CH)]],
                    vbufs[c % 2], vsems[c % 2])

    for c in range(E1W // CH):
        pltpu.make_async_copy(adj_hbm.at[idx1.at[pl.ds(c * CH, CH)]],
                              adj2.at[pl.ds(c * CH, CH)], sb).wait()

    def _flat2(i, _):
        gidx[pl.ds(UPW * HPAD + i * NSAMP, NSAMP)] = adj2[i, :]
        return _
    lax.fori_loop(0, E1W, _flat2, 0, unroll=8)

    n_chunks = (E2W + UPW * HPAD) // CH
    gpc = CH // NSAMP

    with jax.named_scope("gather_ring"):
        def _step(i, carry):
            for b in range(NB):
                c = i * NB + b
                pltpu.make_async_copy(
                    emb_hbm.at[gidx.at[pl.ds(c * CH, CH)]], bufs[b],
                    sems[b]).wait()
                for r in range(gpc):
                    row = c * gpc + r
                    blocks = _sum_rows(bufs[b], r * NSAMP, NSAMP)
                    for j in range(4):
                        macc[row, pl.ds(j * L, L)] = blocks[j]

                @pl.when(c + NB < n_chunks)
                def _fire(c=c, b=b):
                    _start(c + NB, b)
            return carry
        lax.fori_loop(0, n_chunks // NB, _step, 0)
    pltpu.sync_copy(macc.at[pl.ds(UPW * HPAD // NSAMP, E1W)],
                    m2_out.at[pl.ds(wid * E1W, E1W)])

    def _comb(u, carry):
        for j in range(4):
            p0 = macc[4 * u, pl.ds(j * L, L)]
            p1 = macc[4 * u + 1, pl.ds(j * L, L)]
            p2 = macc[4 * u + 2, pl.ds(j * L, L)]
            p3 = macc[4 * u + 3, pl.ds(j * L, L)]
            usum[u, pl.ds(j * L, L)] = (p0 + p1) + (p2 + p3)
        return carry
    lax.fori_loop(0, UPW, _comb, 0, unroll=4)
    pltpu.sync_copy(usum, usum_out.at[pl.ds(base, UPW)])


def _sc_gather(user_pad, item_inputs, n_idxs, adj_item, emb_table):
    mesh = plsc.VectorSubcoreMesh(core_axis_name="c", subcore_axis_name="s",
                                  num_cores=NC, num_subcores=NS)
    f32 = jnp.float32
    return pl.kernel(
        _sc_body,
        out_type=(
            jax.ShapeDtypeStruct((NU, DIM), f32),
            jax.ShapeDtypeStruct((NU, DIM), jnp.float8_e4m3fn),
            jax.ShapeDtypeStruct((NU * NSAMP, DIM), jnp.float8_e4m3fn),
            jax.ShapeDtypeStruct((NU * NSAMP, DIM), f32),
        ),
        mesh=mesh,
        scratch_types=[
            pltpu.VMEM((UPW, HPAD), jnp.int32),
            pltpu.VMEM((UPW, HPAD), jnp.int32),
            pltpu.VMEM((UPW,), jnp.int32),
            pltpu.VMEM((UPW, DIM), jnp.float8_e4m3fn),
            pltpu.VMEM((UPW, NSAMP), jnp.int32),
            pltpu.VMEM((E1W,), jnp.int32),
            pltpu.VMEM((E1W, NSAMP), jnp.int32),
            pltpu.VMEM((E2W + UPW * HPAD,), jnp.int32),
            pltpu.VMEM((CH, DIM), jnp.float8_e4m3fn),
            pltpu.VMEM((CH, DIM), jnp.float8_e4m3fn),
            pltpu.VMEM((CH, DIM), jnp.float8_e4m3fn),
            pltpu.VMEM((CH, DIM), jnp.float8_e4m3fn),
            pltpu.VMEM((CH, DIM), jnp.float8_e4m3fn),
            pltpu.VMEM((CH, DIM), jnp.float8_e4m3fn),
            pltpu.VMEM((UPW, DIM), f32),
            pltpu.VMEM((E1W + UPW * HPAD // NSAMP, DIM), f32),
            pltpu.SemaphoreType.DMA,
            pltpu.SemaphoreType.DMA,
            pltpu.SemaphoreType.DMA,
            pltpu.SemaphoreType.DMA,
            pltpu.SemaphoreType.DMA,
            pltpu.SemaphoreType.DMA,
            pltpu.SemaphoreType.DMA,
            pltpu.SemaphoreType.DMA,
        ],
        compiler_params=pltpu.CompilerParams(use_tc_tiling_on_sc=False,
                                             needs_layout_passes=False),
    )(user_pad, item_inputs, n_idxs, adj_item, emb_table)


RB = 2048
GB = RB // NSAMP


def _tc1_body(v1_ref, m2_ref, pw_ref, pb_ref, m1_ref, a1m_ref):
    v1 = v1_ref[...].astype(jnp.float32) * _INV_S
    m2 = m2_ref[...] * (_INV_S / NSAMP)
    wt = pw_ref[0:DIM, :]
    wb = pw_ref[DIM:2 * DIM, :]
    a1 = jnp.maximum(
        jnp.dot(v1, wt, preferred_element_type=jnp.float32)
        + jnp.dot(m2, wb, preferred_element_type=jnp.float32)
        + pb_ref[...], 0.0)
    rows = lax.broadcasted_iota(jnp.int32, (GB, RB), 0)
    cols = lax.broadcasted_iota(jnp.int32, (GB, RB), 1)
    grp = jnp.where(cols // NSAMP == rows, 1.0 / NSAMP, 0.0)
    m1_ref[...] = jnp.dot(grp, v1, preferred_element_type=jnp.float32)
    a1m_ref[...] = jnp.dot(grp, a1, preferred_element_type=jnp.float32)


def _tc1(v1, m2, pool_W, pool_b):
    n_rows = v1.shape[0]
    grid = (n_rows // RB,)
    return pl.pallas_call(
        _tc1_body,
        grid=grid,
        in_specs=[
            pl.BlockSpec((RB, DIM), lambda i: (i, 0)),
            pl.BlockSpec((RB, DIM), lambda i: (i, 0)),
            pl.BlockSpec((2 * DIM, DIM), lambda i: (0, 0)),
            pl.BlockSpec((1, DIM), lambda i: (0, 0)),
        ],
        out_specs=[
            pl.BlockSpec((GB, DIM), lambda i: (i, 0)),
            pl.BlockSpec((GB, DIM), lambda i: (i, 0)),
        ],
        out_shape=[
            jax.ShapeDtypeStruct((n_rows // NSAMP, DIM), jnp.float32),
            jax.ShapeDtypeStruct((n_rows // NSAMP, DIM), jnp.float32),
        ],
    )(v1, m2, pool_W, pool_b)


def _tc2_body(usum_ref, n_ref, v0_ref, m1_ref, a1m_ref, pw_ref, pb_ref,
              w1_ref, b1_ref, w2_ref, b2_ref, out_ref):
    nmax = jnp.max(n_ref[...]).astype(jnp.float32)
    user = usum_ref[...] * (_INV_S / nmax)
    wt = pw_ref[0:DIM, :]
    wb = pw_ref[DIM:2 * DIM, :]
    pb = pb_ref[...]
    a0 = jnp.maximum(
        jnp.dot(v0_ref[...].astype(jnp.float32) * _INV_S, wt,
                preferred_element_type=jnp.float32)
        + jnp.dot(m1_ref[...], wb, preferred_element_type=jnp.float32)
        + pb, 0.0)
    item = jnp.maximum(
        jnp.dot(a0, wt, preferred_element_type=jnp.float32)
        + jnp.dot(a1m_ref[...], wb, preferred_element_type=jnp.float32)
        + pb, 0.0)
    h = jnp.maximum(
        jnp.dot(user, w1_ref[0:DIM, :], preferred_element_type=jnp.float32)
        + jnp.dot(item, w1_ref[DIM:2 * DIM, :],
                  preferred_element_type=jnp.float32)
        + b1_ref[...], 0.0)
    logit = jnp.dot(h, w2_ref[...], preferred_element_type=jnp.float32) \
        + b2_ref[...]
    m = jnp.max(logit)
    e = jnp.exp(logit - m)
    out_ref[...] = e / jnp.sum(e)


def _tc2(usum, n_idxs, v0, m1, a1m, pool_W, pool_b, fc1_W, fc1_b, fc2_W,
         fc2_b):
    return pl.pallas_call(
        _tc2_body,
        out_shape=jax.ShapeDtypeStruct((NU, 1), jnp.float32),
    )(usum, n_idxs, v0, m1, a1m, pool_W, pool_b, fc1_W, fc1_b, fc2_W, fc2_b)


_PERM = np.concatenate([np.arange(0, DIM, 4), np.arange(2, DIM, 4),
                        np.arange(1, DIM, 4), np.arange(3, DIM, 4)])
_SCALE = 512.0
_INV_S = 1.0 / _SCALE


def kernel(user_inputs, item_inputs, n_idxs, adj_item, adj_adam, emb_table,
           pool_W, pool_b, fc1_W, fc1_b, fc2_W, fc2_b):
    del adj_adam
    user_pad = jnp.pad(user_inputs.astype(jnp.int32),
                       ((0, 0), (0, HPAD - HIST)),
                       constant_values=N_ITEM)
    n_rep = jnp.broadcast_to(n_idxs.astype(jnp.int32)[:, None], (NU, HPAD))
    emb8 = (emb_table * _SCALE).astype(jnp.float8_e4m3fn)
    usum, v0, v1, m2 = _sc_gather(user_pad, item_inputs.astype(jnp.int32),
                                  n_rep,
                                  adj_item.astype(jnp.int32), emb8)
    pool_W1 = jnp.concatenate([pool_W[:DIM], pool_W[DIM:][_PERM]], axis=0)
    fc1_Wp = jnp.concatenate([fc1_W[:DIM][_PERM], fc1_W[DIM:]], axis=0)
    m1, a1m = _tc1(v1, m2, pool_W1, pool_b.reshape(1, DIM))
    probs = _tc2(usum, n_idxs.reshape(8, 128).astype(jnp.int32), v0, m1, a1m,
                 pool_W, pool_b.reshape(1, DIM), fc1_Wp,
                 fc1_b.reshape(1, 32), fc2_W, fc2_b.reshape(1, 1))
    return probs.reshape(-1)


# speedup vs baseline: 1.0301x; 1.0301x over previous
import functools

import numpy as np

import jax
import jax.numpy as jnp
from jax import lax
from jax.experimental import pallas as pl
from jax.experimental.pallas import tpu as pltpu
from jax.experimental.pallas import tpu_sc as plsc

N_ITEM = 100000
DIM = 64
NSAMP = 16
NU = 1024
HIST = 50
HPAD = 64
NC, NS, L = 2, 16, 16
NW = NC * NS
UPW = NU // NW
E1W = UPW * NSAMP
E2W = E1W * NSAMP
CH = 128


def _row_f32x4(buf, r):
    a16, b16 = plsc.unpack(buf[r, :], format=plsc.PackFormat.INTERLEAVED,
                           preferred_element_type=jnp.bfloat16)
    aa, ab = plsc.unpack(a16, format=plsc.PackFormat.INTERLEAVED)
    ba, bb = plsc.unpack(b16, format=plsc.PackFormat.INTERLEAVED)
    return (aa, ab, ba, bb)


def _sum_rows(buf, r0, n):
    parts = []
    for p in range(2):
        acc = list(_row_f32x4(buf, r0 + p))
        for k in range(p + 2, n, 2):
            x = _row_f32x4(buf, r0 + k)
            for j in range(4):
                acc[j] = acc[j] + x[j]
        parts.append(acc)
    return [parts[0][j] + parts[1][j] for j in range(4)]


NB = 4


def _sc_body(user_hbm, item_hbm, nidx_hbm, adj_hbm, emb_hbm,
             usum_out, v0_out, v1_out, m2_out,
             uidx, nbuf, ibuf, v0buf, adj1, idx1, adj2, gidx,
             g0, g1, g2, g3, usum, macc, s0, s1, s2, s3, sa, sb):
    bufs = [g0, g1, g2, g3]
    sems = [s0, s1, s2, s3]
    wid = lax.axis_index("s") * NC + lax.axis_index("c")
    base = wid * UPW

    pltpu.sync_copy(user_hbm.at[pl.ds(base, UPW)], uidx)
    pltpu.sync_copy(nidx_hbm.at[pl.ds(base, UPW)], nbuf)
    pltpu.sync_copy(item_hbm.at[pl.ds(base, UPW)], ibuf)

    pltpu.async_copy(emb_hbm.at[ibuf], v0buf, sa)
    pltpu.async_copy(adj_hbm.at[ibuf], adj1, sa)

    lanes = lax.iota(jnp.int32, L)

    def _clamp(u, _):
        for j in range(HPAD // L):
            raw = uidx[u, pl.ds(j * L, L)]
            n_u = nbuf[u, pl.ds(j * L, L)]
            pos = lanes + (j * L)
            gidx[pl.ds(E2W + u * HPAD + j * L, L)] = jnp.where(
                pos < n_u, raw, jnp.int32(N_ITEM))
        return _
    lax.fori_loop(0, UPW, _clamp, 0, unroll=4)

    pltpu.make_async_copy(emb_hbm.at[ibuf], v0buf, sa).wait()
    pltpu.make_async_copy(adj_hbm.at[ibuf], adj1, sa).wait()

    def _flat1(u, _):
        idx1[pl.ds(u * NSAMP, NSAMP)] = adj1[u, :]
        return _
    lax.fori_loop(0, UPW, _flat1, 0, unroll=8)

    for c in range(E1W // CH):
        pltpu.async_copy(adj_hbm.at[idx1.at[pl.ds(c * CH, CH)]],
                         adj2.at[pl.ds(c * CH, CH)], sb)
    pltpu.sync_copy(v0buf, v0_out.at[pl.ds(base, UPW)])

    with jax.named_scope("v1_phase"):
        for c in range(E1W // CH):
            pltpu.async_copy(emb_hbm.at[idx1.at[pl.ds(c * CH, CH)]],
                             bufs[c], sems[c])
        for c in range(E1W // CH):
            pltpu.make_async_copy(emb_hbm.at[idx1.at[pl.ds(c * CH, CH)]],
                                  bufs[c], sems[c]).wait()
            pltpu.sync_copy(bufs[c],
                            v1_out.at[pl.ds(wid * E1W + c * CH, CH)])

    for c in range(E1W // CH):
        pltpu.make_async_copy(adj_hbm.at[idx1.at[pl.ds(c * CH, CH)]],
                              adj2.at[pl.ds(c * CH, CH)], sb).wait()

    def _flat2(i, _):
        gidx[pl.ds(i * NSAMP, NSAMP)] = adj2[i, :]
        return _
    lax.fori_loop(0, E1W, _flat2, 0, unroll=8)

    n_chunks = (E2W + UPW * HPAD) // CH
    gpc = CH // NSAMP

    def _start(c, b):
        pltpu.async_copy(emb_hbm.at[gidx.at[pl.ds(c * CH, CH)]],
                         bufs[b], sems[b])

    with jax.named_scope("gather_ring"):
        for b in range(NB):
            _start(b, b)

        def _step(i, carry):
            for b in range(NB):
                c = i * NB + b
                pltpu.make_async_copy(
                    emb_hbm.at[gidx.at[pl.ds(c * CH, CH)]], bufs[b],
                    sems[b]).wait()
                for r in range(gpc):
                    row = c * gpc + r
                    blocks = _sum_rows(bufs[b], r * NSAMP, NSAMP)
                    for j in range(4):
                        macc[row, pl.ds(j * L, L)] = blocks[j]

                @pl.when(c + NB < n_chunks)
                def _fire(c=c, b=b):
                    _start(c + NB, b)
            return carry
        lax.fori_loop(0, n_chunks // NB, _step, 0)
    pltpu.sync_copy(macc.at[pl.ds(0, E1W)], m2_out.at[pl.ds(wid * E1W, E1W)])

    def _comb(u, carry):
        for j in range(4):
            p0 = macc[E1W + 4 * u, pl.ds(j * L, L)]
            p1 = macc[E1W + 4 * u + 1, pl.ds(j * L, L)]
            p2 = macc[E1W + 4 * u + 2, pl.ds(j * L, L)]
            p3 = macc[E1W + 4 * u + 3, pl.ds(j * L, L)]
            usum[u, pl.ds(j * L, L)] = (p0 + p1) + (p2 + p3)
        return carry
    lax.fori_loop(0, UPW, _comb, 0, unroll=4)
    pltpu.sync_copy(usum, usum_out.at[pl.ds(base, UPW)])


def _sc_gather(user_pad, item_inputs, n_idxs, adj_item, emb_table):
    mesh = plsc.VectorSubcoreMesh(core_axis_name="c", subcore_axis_name="s",
                                  num_cores=NC, num_subcores=NS)
    f32 = jnp.float32
    return pl.kernel(
        _sc_body,
        out_type=(
            jax.ShapeDtypeStruct((NU, DIM), f32),
            jax.ShapeDtypeStruct((NU, DIM), jnp.float8_e4m3fn),
            jax.ShapeDtypeStruct((NU * NSAMP, DIM), jnp.float8_e4m3fn),
            jax.ShapeDtypeStruct((NU * NSAMP, DIM), f32),
        ),
        mesh=mesh,
        scratch_types=[
            pltpu.VMEM((UPW, HPAD), jnp.int32),
            pltpu.VMEM((UPW, HPAD), jnp.int32),
            pltpu.VMEM((UPW,), jnp.int32),
            pltpu.VMEM((UPW, DIM), jnp.float8_e4m3fn),
            pltpu.VMEM((UPW, NSAMP), jnp.int32),
            pltpu.VMEM((E1W,), jnp.int32),
            pltpu.VMEM((E1W, NSAMP), jnp.int32),
            pltpu.VMEM((E2W + UPW * HPAD,), jnp.int32),
            pltpu.VMEM((CH, DIM), jnp.float8_e4m3fn),
            pltpu.VMEM((CH, DIM), jnp.float8_e4m3fn),
            pltpu.VMEM((CH, DIM), jnp.float8_e4m3fn),
            pltpu.VMEM((CH, DIM), jnp.float8_e4m3fn),
            pltpu.VMEM((UPW, DIM), f32),
            pltpu.VMEM((E1W + UPW * HPAD // NSAMP, DIM), f32),
            pltpu.SemaphoreType.DMA,
            pltpu.SemaphoreType.DMA,
            pltpu.SemaphoreType.DMA,
            pltpu.SemaphoreType.DMA,
            pltpu.SemaphoreType.DMA,
            pltpu.SemaphoreType.DMA,
        ],
        compiler_params=pltpu.CompilerParams(use_tc_tiling_on_sc=False,
                                             needs_layout_passes=False),
    )(user_pad, item_inputs, n_idxs, adj_item, emb_table)


RB = 2048
GB = RB // NSAMP


def _tc1_body(v1_ref, m2_ref, pw_ref, pb_ref, m1_ref, a1m_ref):
    v1 = v1_ref[...].astype(jnp.float32) * _INV_S
    m2 = m2_ref[...] * (_INV_S / NSAMP)
    wt = pw_ref[0:DIM, :]
    wb = pw_ref[DIM:2 * DIM, :]
    a1 = jnp.maximum(
        jnp.dot(v1, wt, preferred_element_type=jnp.float32)
        + jnp.dot(m2, wb, preferred_element_type=jnp.float32)
        + pb_ref[...], 0.0)
    rows = lax.broadcasted_iota(jnp.int32, (GB, RB), 0)
    cols = lax.broadcasted_iota(jnp.int32, (GB, RB), 1)
    grp = jnp.where(cols // NSAMP == rows, 1.0 / NSAMP, 0.0)
    m1_ref[...] = jnp.dot(grp, v1, preferred_element_type=jnp.float32)
    a1m_ref[...] = jnp.dot(grp, a1, preferred_element_type=jnp.float32)


def _tc1(v1, m2, pool_W, pool_b):
    n_rows = v1.shape[0]
    grid = (n_rows // RB,)
    return pl.pallas_call(
        _tc1_body,
        grid=grid,
        in_specs=[
            pl.BlockSpec((RB, DIM), lambda i: (i, 0)),
            pl.BlockSpec((RB, DIM), lambda i: (i, 0)),
            pl.BlockSpec((2 * DIM, DIM), lambda i: (0, 0)),
            pl.BlockSpec((1, DIM), lambda i: (0, 0)),
        ],
        out_specs=[
            pl.BlockSpec((GB, DIM), lambda i: (i, 0)),
            pl.BlockSpec((GB, DIM), lambda i: (i, 0)),
        ],
        out_shape=[
            jax.ShapeDtypeStruct((n_rows // NSAMP, DIM), jnp.float32),
            jax.ShapeDtypeStruct((n_rows // NSAMP, DIM), jnp.float32),
        ],
    )(v1, m2, pool_W, pool_b)


def _tc2_body(usum_ref, n_ref, v0_ref, m1_ref, a1m_ref, pw_ref, pb_ref,
              w1_ref, b1_ref, w2_ref, b2_ref, out_ref):
    nmax = jnp.max(n_ref[...]).astype(jnp.float32)
    user = usum_ref[...] * (_INV_S / nmax)
    wt = pw_ref[0:DIM, :]
    wb = pw_ref[DIM:2 * DIM, :]
    pb = pb_ref[...]
    a0 = jnp.maximum(
        jnp.dot(v0_ref[...].astype(jnp.float32) * _INV_S, wt,
                preferred_element_type=jnp.float32)
        + jnp.dot(m1_ref[...], wb, preferred_element_type=jnp.float32)
        + pb, 0.0)
    item = jnp.maximum(
        jnp.dot(a0, wt, preferred_element_type=jnp.float32)
        + jnp.dot(a1m_ref[...], wb, preferred_element_type=jnp.float32)
        + pb, 0.0)
    h = jnp.maximum(
        jnp.dot(user, w1_ref[0:DIM, :], preferred_element_type=jnp.float32)
        + jnp.dot(item, w1_ref[DIM:2 * DIM, :],
                  preferred_element_type=jnp.float32)
        + b1_ref[...], 0.0)
    logit = jnp.dot(h, w2_ref[...], preferred_element_type=jnp.float32) \
        + b2_ref[...]
    m = jnp.max(logit)
    e = jnp.exp(logit - m)
    out_ref[...] = e / jnp.sum(e)


def _tc2(usum, n_idxs, v0, m1, a1m, pool_W, pool_b, fc1_W, fc1_b, fc2_W,
         fc2_b):
    return pl.pallas_call(
        _tc2_body,
        out_shape=jax.ShapeDtypeStruct((NU, 1), jnp.float32),
    )(usum, n_idxs, v0, m1, a1m, pool_W, pool_b, fc1_W, fc1_b, fc2_W, fc2_b)


_PERM = np.concatenate([np.arange(0, DIM, 4), np.arange(2, DIM, 4),
                        np.arange(1, DIM, 4), np.arange(3, DIM, 4)])
_SCALE = 512.0
_INV_S = 1.0 / _SCALE


def kernel(user_inputs, item_inputs, n_idxs, adj_item, adj_adam, emb_table,
           pool_W, pool_b, fc1_W, fc1_b, fc2_W, fc2_b):
    del adj_adam
    user_pad = jnp.pad(user_inputs.astype(jnp.int32),
                       ((0, 0), (0, HPAD - HIST)),
                       constant_values=N_ITEM)
    n_rep = jnp.broadcast_to(n_idxs.astype(jnp.int32)[:, None], (NU, HPAD))
    emb8 = (emb_table * _SCALE).astype(jnp.float8_e4m3fn)
    usum, v0, v1, m2 = _sc_gather(user_pad, item_inputs.astype(jnp.int32),
                                  n_rep,
                                  adj_item.astype(jnp.int32), emb8)
    pool_W1 = jnp.concatenate([pool_W[:DIM], pool_W[DIM:][_PERM]], axis=0)
    fc1_Wp = jnp.concatenate([fc1_W[:DIM][_PERM], fc1_W[DIM:]], axis=0)
    m1, a1m = _tc1(v1, m2, pool_W1, pool_b.reshape(1, DIM))
    probs = _tc2(usum, n_idxs.reshape(8, 128).astype(jnp.int32), v0, m1, a1m,
                 pool_W, pool_b.reshape(1, DIM), fc1_Wp,
                 fc1_b.reshape(1, 32), fc2_W, fc2_b.reshape(1, 1))
    return probs.reshape(-1)
